# baseline (speedup 1.0000x reference)
import functools

import jax
import jax.numpy as jnp
from jax import lax
from jax.experimental import pallas as pl
from jax.experimental.pallas import tpu as pltpu
from jax.experimental.pallas import tpu_sc as plsc

_BATCH = 16384
_N_WORDS = 50
_TOKEN_SPACE = 100000
_N_OOV = 100
_V = 1001
_VPAD = 1008
_L = 16
_NC = 2
_NS = 16
_NW = _NC * _NS
_COLS_PER_W = _BATCH // _NW
_CCOLS = 128
_NCHUNK = _COLS_PER_W // _CCOLS
_INIT_UNROLL = 10

_mesh = plsc.VectorSubcoreMesh(core_axis_name="c", subcore_axis_name="s")


@functools.partial(
    pl.kernel,
    mesh=_mesh,
    out_type=jax.ShapeDtypeStruct((_N_WORDS, _BATCH), jnp.uint32),
    scratch_types=[
        pltpu.VMEM((_TOKEN_SPACE,), jnp.int32),
        pltpu.VMEM((_VPAD,), jnp.int32),
        pltpu.VMEM((_N_WORDS, _CCOLS), jnp.int32),
        pltpu.VMEM((_N_WORDS, _CCOLS), jnp.int32),
        pltpu.VMEM((_N_WORDS, _CCOLS), jnp.uint32),
        pltpu.VMEM((_N_WORDS, _CCOLS), jnp.uint32),
        pltpu.SemaphoreType.DMA,
        pltpu.SemaphoreType.DMA,
        pltpu.SemaphoreType.DMA,
        pltpu.SemaphoreType.DMA,
    ],
    compiler_params=pltpu.CompilerParams(needs_layout_passes=False),
)
def _lookup(tok_hbm, keys_hbm, out_hbm, table, keys_v, tok_a, tok_b, res_a,
            res_b, sem_a, sem_b, osem_a, osem_b):
    wid = lax.axis_index("s") * _NC + lax.axis_index("c")
    col0 = wid * _COLS_PER_W

    tok_bufs = (tok_a, tok_b)
    sems = (sem_a, sem_b)
    res_bufs = (res_a, res_b)
    osems = (osem_a, osem_b)

    copies = [None] * _NCHUNK
    copies[0] = pltpu.make_async_copy(
        tok_hbm.at[:, pl.ds(col0, _CCOLS)], tok_a, sem_a
    )
    copies[0].start()

    pltpu.sync_copy(keys_hbm, keys_v.at[pl.ds(0, _V)])

    iota = lax.iota(jnp.int32, _L)

    @plsc.parallel_loop(jnp.int32(0), jnp.int32(_TOKEN_SPACE // _L),
                        jnp.int32(1), unroll=_INIT_UNROLL)
    def _(i):
        off = i * jnp.int32(_L)
        base = jnp.int32(_V) + lax.rem(off, jnp.int32(_N_OOV))
        rv = base + iota
        rv = jnp.where(rv >= jnp.int32(_V + _N_OOV),
                       rv - jnp.int32(_N_OOV), rv)
        table[pl.ds(off, _L)] = rv

    @plsc.parallel_loop(jnp.int32(0), jnp.int32(_VPAD // _L), jnp.int32(1),
                        unroll=4)
    def _(j):
        off = j * jnp.int32(_L)
        lanes = iota + off
        keys = keys_v[pl.ds(off, _L)]
        plsc.store_scatter(table, [keys], lanes, mask=lanes < jnp.int32(_V))

    out_copies = [None] * _NCHUNK
    for c in range(_NCHUNK):
        tok_v = tok_bufs[c % 2]
        res_v = res_bufs[c % 2]
        copies[c].wait()
        if c + 1 < _NCHUNK:
            copies[c + 1] = pltpu.make_async_copy(
                tok_hbm.at[:, pl.ds(col0 + (c + 1) * _CCOLS, _CCOLS)],
                tok_bufs[(c + 1) % 2],
                sems[(c + 1) % 2],
            )
            copies[c + 1].start()
        if c >= 2:
            out_copies[c - 2].wait()

        @plsc.parallel_loop(jnp.int32(0),
                            jnp.int32(_N_WORDS * (_CCOLS // _L)),
                            jnp.int32(1), unroll=8)
        def _(i):
            w = lax.div(i, jnp.int32(_CCOLS // _L))
            kb16 = lax.rem(i, jnp.int32(_CCOLS // _L)) * jnp.int32(_L)
            t = tok_v[w, pl.ds(kb16, _L)]
            g = plsc.load_gather(table, [t])
            res_v[w, pl.ds(kb16, _L)] = plsc.bitcast(g, jnp.uint32)

        out_copies[c] = pltpu.make_async_copy(
            res_v, out_hbm.at[:, pl.ds(col0 + c * _CCOLS, _CCOLS)], osems[c % 2]
        )
        out_copies[c].start()

    out_copies[_NCHUNK - 2].wait()
    out_copies[_NCHUNK - 1].wait()


def kernel(inputs, vocab_keys):
    tok = inputs.astype(jnp.int32).T
    keys = vocab_keys.astype(jnp.int32)
    return _lookup(tok, keys).T.astype(jnp.int64)


# speedup vs baseline: 1.0843x; 1.0843x over previous
import functools

import jax
import jax.numpy as jnp
from jax import lax
from jax.experimental import pallas as pl
from jax.experimental.pallas import tpu as pltpu
from jax.experimental.pallas import tpu_sc as plsc

_BATCH = 16384
_N_WORDS = 50
_TOKEN_SPACE = 100000
_N_OOV = 100
_V = 1001
_VPAD = 1008
_L = 16
_NC = 2
_NS = 16
_NW = _NC * _NS
_COLS_PER_W = _BATCH // _NW
_CCOLS = 128
_NCHUNK = _COLS_PER_W // _CCOLS
_INIT_UNROLL = 25

_mesh = plsc.VectorSubcoreMesh(core_axis_name="c", subcore_axis_name="s")


@functools.partial(
    pl.kernel,
    mesh=_mesh,
    out_type=jax.ShapeDtypeStruct((_N_WORDS, _BATCH), jnp.uint32),
    scratch_types=[
        pltpu.VMEM((_TOKEN_SPACE,), jnp.int32),
        pltpu.VMEM((_VPAD,), jnp.int32),
        pltpu.VMEM((_N_WORDS, _CCOLS), jnp.int32),
        pltpu.VMEM((_N_WORDS, _CCOLS), jnp.int32),
        pltpu.VMEM((_N_WORDS, _CCOLS), jnp.uint32),
        pltpu.VMEM((_N_WORDS, _CCOLS), jnp.uint32),
        pltpu.SemaphoreType.DMA,
        pltpu.SemaphoreType.DMA,
        pltpu.SemaphoreType.DMA,
        pltpu.SemaphoreType.DMA,
    ],
    compiler_params=pltpu.CompilerParams(needs_layout_passes=False),
)
def _lookup(tok_hbm, keys_hbm, out_hbm, table, keys_v, tok_a, tok_b, res_a,
            res_b, sem_a, sem_b, osem_a, osem_b):
    wid = lax.axis_index("s") * _NC + lax.axis_index("c")
    col0 = wid * _COLS_PER_W

    tok_bufs = (tok_a, tok_b)
    sems = (sem_a, sem_b)
    res_bufs = (res_a, res_b)
    osems = (osem_a, osem_b)

    copies = [None] * _NCHUNK
    copies[0] = pltpu.make_async_copy(
        tok_hbm.at[:, pl.ds(col0, _CCOLS)], tok_a, sem_a
    )
    copies[0].start()

    pltpu.sync_copy(keys_hbm, keys_v.at[pl.ds(0, _V)])

    iota = lax.iota(jnp.int32, _L)

    @plsc.parallel_loop(jnp.int32(0), jnp.int32(_TOKEN_SPACE // _L),
                        jnp.int32(1), unroll=_INIT_UNROLL)
    def _(i):
        off = i * jnp.int32(_L)
        base = jnp.int32(_V) + lax.rem(off, jnp.int32(_N_OOV))
        rv = base + iota
        rv = jnp.where(rv >= jnp.int32(_V + _N_OOV),
                       rv - jnp.int32(_N_OOV), rv)
        table[pl.ds(off, _L)] = rv

    @plsc.parallel_loop(jnp.int32(0), jnp.int32(_VPAD // _L), jnp.int32(1),
                        unroll=4)
    def _(j):
        off = j * jnp.int32(_L)
        lanes = iota + off
        keys = keys_v[pl.ds(off, _L)]
        plsc.store_scatter(table, [keys], lanes, mask=lanes < jnp.int32(_V))

    out_copies = [None] * _NCHUNK
    for c in range(_NCHUNK):
        tok_v = tok_bufs[c % 2]
        res_v = res_bufs[c % 2]
        copies[c].wait()
        if c + 1 < _NCHUNK:
            copies[c + 1] = pltpu.make_async_copy(
                tok_hbm.at[:, pl.ds(col0 + (c + 1) * _CCOLS, _CCOLS)],
                tok_bufs[(c + 1) % 2],
                sems[(c + 1) % 2],
            )
            copies[c + 1].start()
        if c >= 2:
            out_copies[c - 2].wait()

        @plsc.parallel_loop(jnp.int32(0),
                            jnp.int32(_N_WORDS * (_CCOLS // _L)),
                            jnp.int32(1), unroll=16)
        def _(i):
            w = lax.div(i, jnp.int32(_CCOLS // _L))
            kb16 = lax.rem(i, jnp.int32(_CCOLS // _L)) * jnp.int32(_L)
            t = tok_v[w, pl.ds(kb16, _L)]
            g = plsc.load_gather(table, [t])
            res_v[w, pl.ds(kb16, _L)] = plsc.bitcast(g, jnp.uint32)

        out_copies[c] = pltpu.make_async_copy(
            res_v, out_hbm.at[:, pl.ds(col0 + c * _CCOLS, _CCOLS)], osems[c % 2]
        )
        out_copies[c].start()

    out_copies[_NCHUNK - 2].wait()
    out_copies[_NCHUNK - 1].wait()


def kernel(inputs, vocab_keys):
    tok = inputs.astype(jnp.int32).T
    keys = vocab_keys.astype(jnp.int32)
    return _lookup(tok, keys).T.astype(jnp.int64)
